# baseline (speedup 1.0000x reference)
import functools

import jax
import jax.numpy as jnp
from jax import lax
from jax.experimental import pallas as pl
from jax.experimental.pallas import tpu as pltpu
from jax.experimental.pallas import tpu_sc as plsc

B, T, DXN, RXN = 128, 20, 40, 40
D = 256
DXV, RXV = 4880, 3000
ATTN = 128
DPL = 4880
SEG = B * T

NC, NS = 2, 16
NW = NC * NS
SEGW = SEG // NW


_CH = 2
_NCHUNK = SEGW // _CH
_ROWS = _CH * DXN


def _make_bag():
  mesh = plsc.VectorSubcoreMesh(core_axis_name="c", subcore_axis_name="s")

  @functools.partial(
      pl.kernel,
      mesh=mesh,
      out_type=jax.ShapeDtypeStruct((SEG, D), jnp.float32),
      compiler_params=pltpu.CompilerParams(use_tc_tiling_on_sc=False),
      scratch_types=[
          pltpu.VMEM((SEGW * DXN,), jnp.int32),
          pltpu.VMEM((SEGW * RXN,), jnp.int32),
          pltpu.VMEM((2, _ROWS, D), jnp.float32),
          pltpu.VMEM((2, _ROWS, D), jnp.float32),
          pltpu.VMEM((SEGW, D), jnp.float32),
          pltpu.SemaphoreType.DMA,
          pltpu.SemaphoreType.DMA,
          pltpu.SemaphoreType.DMA,
          pltpu.SemaphoreType.DMA,
      ],
  )
  def bag(dxi_hbm, rxi_hbm, dxemb_hbm, rxemb_hbm, out_hbm,
          dxi_v, rxi_v, rows_dx, rows_rx, acc_v, sdx0, sdx1, srx0, srx1):
    wid = lax.axis_index("s") * NC + lax.axis_index("c")
    base = wid * SEGW
    pltpu.sync_copy(dxi_hbm.at[pl.ds(base * DXN, SEGW * DXN)], dxi_v)
    pltpu.sync_copy(rxi_hbm.at[pl.ds(base * RXN, SEGW * RXN)], rxi_v)
    sems = ((sdx0, srx0), (sdx1, srx1))

    def issue(c, slot):
      sd, sr = sems[slot]
      pltpu.async_copy(
          dxemb_hbm.at[dxi_v.at[pl.ds(c * _ROWS, _ROWS)]], rows_dx.at[slot], sd)
      pltpu.async_copy(
          rxemb_hbm.at[rxi_v.at[pl.ds(c * _ROWS, _ROWS)]], rows_rx.at[slot], sr)

    def consume(c, slot):
      def seg_body(k, carry):
        def row_body(r, acc):
          return tuple(
              acc[j]
              + rows_dx[slot, r, pl.ds(16 * j, 16)]
              + rows_rx[slot, r, pl.ds(16 * j, 16)]
              for j in range(D // 16))

        zeros = tuple(jnp.zeros((16,), jnp.float32) for _ in range(D // 16))
        acc = lax.fori_loop(k * DXN, (k + 1) * DXN, row_body, zeros)
        for j in range(D // 16):
          acc_v[c * _CH + k, pl.ds(16 * j, 16)] = acc[j]
        return carry

      lax.fori_loop(0, _CH, seg_body, 0)

    def drain(slot):
      sd, sr = sems[slot]
      pltpu.make_async_copy(
          dxemb_hbm.at[dxi_v.at[pl.ds(0, _ROWS)]], rows_dx.at[slot], sd).wait()
      pltpu.make_async_copy(
          rxemb_hbm.at[rxi_v.at[pl.ds(0, _ROWS)]], rows_rx.at[slot], sr).wait()

    issue(0, 0)

    def pair_body(i, carry):
      c0 = 2 * i
      issue(c0 + 1, 1)
      drain(0)
      consume(c0, 0)

      @pl.when(c0 + 2 < _NCHUNK)
      def _():
        issue(c0 + 2, 0)

      drain(1)
      consume(c0 + 1, 1)
      return carry

    lax.fori_loop(0, _NCHUNK // 2, pair_body, 0)
    pltpu.sync_copy(acc_v, out_hbm.at[pl.ds(base, SEGW)])

  return bag


_bag = _make_bag()


def _attn_body(pooled_ref, attn_w_ref, attn_b_ref, attnc_w_ref, attnc_b_ref,
               dp_w_ref, dp_b_ref, read_w_ref, read_b_ref, mort_w_ref,
               mort_b_ref, visit_out, dp_out, read_out, mort_out):
  visit = jnp.tanh(pooled_ref[...].astype(jnp.float32))
  v3 = visit.reshape(B, T, D)
  last = v3[:, T - 1, :]
  w1 = attn_w_ref[0:D, :]
  w2 = attn_w_ref[D:2 * D, :]
  h = jnp.dot(visit, w1, preferred_element_type=jnp.float32)
  h2 = jnp.dot(last, w2, preferred_element_type=jnp.float32)
  e = jnp.tanh(h.reshape(B, T, ATTN) + h2[:, None, :] + attn_b_ref[...])
  sc = jnp.sum(e * attnc_w_ref[...][None, :, :], axis=-1) + attnc_b_ref[0, 0]
  m = jnp.max(sc, axis=1, keepdims=True)
  a = jnp.exp(sc - m)
  alpha = a / jnp.sum(a, axis=1, keepdims=True)
  pt = jnp.sum(alpha[:, :, None] * v3, axis=1)
  dp = jax.nn.sigmoid(
      jnp.dot(pt, dp_w_ref[...], preferred_element_type=jnp.float32)
      + dp_b_ref[...])
  rd = jax.nn.sigmoid(
      jnp.sum(pt * read_w_ref[...], axis=-1, keepdims=True) + read_b_ref[0, 0])
  mt = jax.nn.sigmoid(
      jnp.sum(pt * mort_w_ref[...], axis=-1, keepdims=True) + mort_b_ref[0, 0])
  visit_out[...] = visit.astype(jnp.bfloat16)
  dp_out[...] = dp
  read_out[...] = rd
  mort_out[...] = mt


def _attn_call(pooled, attn_w, attn_b, attnc_w, attnc_b, dp_w, dp_b,
               read_w, read_b, mort_w, mort_b):
  return pl.pallas_call(
      _attn_body,
      out_shape=(
          jax.ShapeDtypeStruct((SEG, D), jnp.bfloat16),
          jax.ShapeDtypeStruct((B, DPL), jnp.float32),
          jax.ShapeDtypeStruct((B, 1), jnp.float32),
          jax.ShapeDtypeStruct((B, 1), jnp.float32),
      ),
  )(pooled, attn_w, attn_b, attnc_w, attnc_b, dp_w, dp_b, read_w, read_b,
    mort_w, mort_b)


_VT = 512


def _loss_body(num_tiles, vocab, visit_ref, w_ref, b_ref, lab_ref,
               z_out, s1_out, sl_out):
  i = pl.program_id(0)

  @pl.when(i == 0)
  def _():
    z_out[...] = jnp.zeros_like(z_out)
    s1_out[...] = jnp.zeros_like(s1_out)
    sl_out[...] = jnp.zeros_like(sl_out)

  logits = (jnp.dot(visit_ref[...], w_ref[...],
                    preferred_element_type=jnp.float32) + b_ref[...])
  col = i * _VT + lax.broadcasted_iota(jnp.int32, (1, _VT), 1)
  valid = col < vocab
  lab = lab_ref[...].reshape(SEG, _VT)
  expl = jnp.where(valid, jnp.exp(logits), 0.0)
  s1 = jnp.where(valid, lab * logits, 0.0)
  sl = jnp.where(valid, lab, 0.0)
  z_out[...] += jnp.sum(expl, axis=1, keepdims=True)
  s1_out[...] += jnp.sum(s1, axis=1, keepdims=True)
  sl_out[...] += jnp.sum(sl, axis=1, keepdims=True)


def _loss_call(visit_bf, w_bf, b2, labels, vocab):
  num_tiles = pl.cdiv(vocab, _VT)
  one = jax.ShapeDtypeStruct((SEG, 1), jnp.float32)
  return pl.pallas_call(
      functools.partial(_loss_body, num_tiles, vocab),
      grid=(num_tiles,),
      in_specs=[
          pl.BlockSpec((SEG, D), lambda i: (0, 0)),
          pl.BlockSpec((D, _VT), lambda i: (0, i)),
          pl.BlockSpec((1, _VT), lambda i: (0, i)),
          pl.BlockSpec((B, T, _VT), lambda i: (0, 0, i)),
      ],
      out_specs=(
          pl.BlockSpec((SEG, 1), lambda i: (0, 0)),
          pl.BlockSpec((SEG, 1), lambda i: (0, 0)),
          pl.BlockSpec((SEG, 1), lambda i: (0, 0)),
      ),
      out_shape=(one, one, one),
  )(visit_bf, w_bf, b2, labels)


def _combine_body(z1, s11, sl1, z2, s12, sl2, out_ref):
  z = z1[...] + z2[...]
  s1 = s11[...] + s12[...]
  sl = sl1[...] + sl2[...]
  loss_rows = s1 - jnp.log(z) * sl
  out_ref[...] = -jnp.sum(loss_rows, axis=0, keepdims=True) / B


def _combine_call(pdx, prx):
  return pl.pallas_call(
      _combine_body,
      out_shape=jax.ShapeDtypeStruct((1, 1), jnp.float32),
  )(*pdx, *prx)


def kernel(dxseqs, drugseqs, dx_onehot, drug_onehot, EHRdxEmb, EHRdrugEmb,
           attn_W, attn_b, attnC_W, attnC_b, dp_W, dp_b, read_W, read_b,
           mort_W, mort_b, co_W, co_b):
  dxi = dxseqs.reshape(-1).astype(jnp.int32)
  rxi = drugseqs.reshape(-1).astype(jnp.int32)
  pooled = _bag(dxi, rxi, EHRdxEmb, EHRdrugEmb)

  visit_bf, dpPred, readPred, mortPred = _attn_call(
      pooled, attn_W, attn_b.reshape(1, ATTN), attnC_W.reshape(1, ATTN),
      attnC_b.reshape(1, 1), dp_W, dp_b.reshape(1, DPL),
      read_W.reshape(1, D), read_b.reshape(1, 1), mort_W.reshape(1, D),
      mort_b.reshape(1, 1))

  co_Wb = co_W.astype(jnp.bfloat16)
  pdx = _loss_call(visit_bf, co_Wb[:, :DXV], co_b[:DXV].reshape(1, DXV),
                   dx_onehot, DXV)
  prx = _loss_call(visit_bf, co_Wb[:, DXV:], co_b[DXV:].reshape(1, RXV),
                   drug_onehot, RXV)
  co_loss = _combine_call(pdx, prx)[0, 0]
  return dpPred, readPred, mortPred, co_loss


# speedup vs baseline: 1.0330x; 1.0330x over previous
import functools

import jax
import jax.numpy as jnp
from jax import lax
from jax.experimental import pallas as pl
from jax.experimental.pallas import tpu as pltpu
from jax.experimental.pallas import tpu_sc as plsc

B, T, DXN, RXN = 128, 20, 40, 40
D = 256
DXV, RXV = 4880, 3000
ATTN = 128
DPL = 4880
SEG = B * T

NC, NS = 2, 16
NW = NC * NS
SEGW = SEG // NW


_CH = 2
_NCHUNK = SEGW // _CH
_ROWS = _CH * DXN


def _make_bag():
  mesh = plsc.VectorSubcoreMesh(core_axis_name="c", subcore_axis_name="s")

  @functools.partial(
      pl.kernel,
      mesh=mesh,
      out_type=jax.ShapeDtypeStruct((SEG, D), jnp.float32),
      scratch_types=[
          pltpu.VMEM((SEGW * DXN,), jnp.int32),
          pltpu.VMEM((SEGW * RXN,), jnp.int32),
          pltpu.VMEM((2, _ROWS, D), jnp.float32),
          pltpu.VMEM((2, _ROWS, D), jnp.float32),
          pltpu.VMEM((SEGW, D), jnp.float32),
          pltpu.SemaphoreType.DMA,
          pltpu.SemaphoreType.DMA,
          pltpu.SemaphoreType.DMA,
          pltpu.SemaphoreType.DMA,
      ],
  )
  def bag(dxi_hbm, rxi_hbm, dxemb_hbm, rxemb_hbm, out_hbm,
          dxi_v, rxi_v, rows_dx, rows_rx, acc_v, sdx0, sdx1, srx0, srx1):
    wid = lax.axis_index("s") * NC + lax.axis_index("c")
    base = wid * SEGW
    pltpu.sync_copy(dxi_hbm.at[pl.ds(base * DXN, SEGW * DXN)], dxi_v)
    pltpu.sync_copy(rxi_hbm.at[pl.ds(base * RXN, SEGW * RXN)], rxi_v)
    sems = ((sdx0, srx0), (sdx1, srx1))

    def issue(c, slot):
      sd, sr = sems[slot]
      pltpu.async_copy(
          dxemb_hbm.at[dxi_v.at[pl.ds(c * _ROWS, _ROWS)]], rows_dx.at[slot], sd)
      pltpu.async_copy(
          rxemb_hbm.at[rxi_v.at[pl.ds(c * _ROWS, _ROWS)]], rows_rx.at[slot], sr)

    def consume(c, slot):
      def seg_body(k, carry):
        def row_body(h, acc):
          r = 2 * h
          return tuple(
              acc[j]
              + (rows_dx[slot, r, pl.ds(16 * j, 16)]
                 + rows_rx[slot, r, pl.ds(16 * j, 16)])
              + (rows_dx[slot, r + 1, pl.ds(16 * j, 16)]
                 + rows_rx[slot, r + 1, pl.ds(16 * j, 16)])
              for j in range(D // 16))

        zeros = tuple(jnp.zeros((16,), jnp.float32) for _ in range(D // 16))
        acc = lax.fori_loop(k * DXN // 2, (k + 1) * DXN // 2, row_body, zeros)
        for j in range(D // 16):
          acc_v[c * _CH + k, pl.ds(16 * j, 16)] = acc[j]
        return carry

      lax.fori_loop(0, _CH, seg_body, 0)

    def drain(slot):
      sd, sr = sems[slot]
      pltpu.make_async_copy(
          dxemb_hbm.at[dxi_v.at[pl.ds(0, _ROWS)]], rows_dx.at[slot], sd).wait()
      pltpu.make_async_copy(
          rxemb_hbm.at[rxi_v.at[pl.ds(0, _ROWS)]], rows_rx.at[slot], sr).wait()

    issue(0, 0)

    def pair_body(i, carry):
      c0 = 2 * i
      issue(c0 + 1, 1)
      drain(0)
      consume(c0, 0)

      @pl.when(c0 + 2 < _NCHUNK)
      def _():
        issue(c0 + 2, 0)

      drain(1)
      consume(c0 + 1, 1)
      return carry

    lax.fori_loop(0, _NCHUNK // 2, pair_body, 0)
    pltpu.sync_copy(acc_v, out_hbm.at[pl.ds(base, SEGW)])

  return bag


_bag = _make_bag()


def _attn_body(pooled_ref, attn_w_ref, attn_b_ref, attnc_w_ref, attnc_b_ref,
               dp_w_ref, dp_b_ref, read_w_ref, read_b_ref, mort_w_ref,
               mort_b_ref, visit_out, dp_out, read_out, mort_out):
  visit = jnp.tanh(pooled_ref[...].astype(jnp.float32))
  v3 = visit.reshape(B, T, D)
  last = v3[:, T - 1, :]
  w1 = attn_w_ref[0:D, :]
  w2 = attn_w_ref[D:2 * D, :]
  h = jnp.dot(visit, w1, preferred_element_type=jnp.float32)
  h2 = jnp.dot(last, w2, preferred_element_type=jnp.float32)
  e = jnp.tanh(h.reshape(B, T, ATTN) + h2[:, None, :] + attn_b_ref[...])
  sc = jnp.sum(e * attnc_w_ref[...][None, :, :], axis=-1) + attnc_b_ref[0, 0]
  m = jnp.max(sc, axis=1, keepdims=True)
  a = jnp.exp(sc - m)
  alpha = a / jnp.sum(a, axis=1, keepdims=True)
  pt = jnp.sum(alpha[:, :, None] * v3, axis=1)
  dp = jax.nn.sigmoid(
      jnp.dot(pt, dp_w_ref[...], preferred_element_type=jnp.float32)
      + dp_b_ref[...])
  rd = jax.nn.sigmoid(
      jnp.sum(pt * read_w_ref[...], axis=-1, keepdims=True) + read_b_ref[0, 0])
  mt = jax.nn.sigmoid(
      jnp.sum(pt * mort_w_ref[...], axis=-1, keepdims=True) + mort_b_ref[0, 0])
  visit_out[...] = visit.astype(jnp.bfloat16)
  dp_out[...] = dp
  read_out[...] = rd
  mort_out[...] = mt


def _attn_call(pooled, attn_w, attn_b, attnc_w, attnc_b, dp_w, dp_b,
               read_w, read_b, mort_w, mort_b):
  return pl.pallas_call(
      _attn_body,
      out_shape=(
          jax.ShapeDtypeStruct((SEG, D), jnp.bfloat16),
          jax.ShapeDtypeStruct((B, DPL), jnp.float32),
          jax.ShapeDtypeStruct((B, 1), jnp.float32),
          jax.ShapeDtypeStruct((B, 1), jnp.float32),
      ),
  )(pooled, attn_w, attn_b, attnc_w, attnc_b, dp_w, dp_b, read_w, read_b,
    mort_w, mort_b)


_VT = 512


def _loss_body(num_tiles, vocab, visit_ref, w_ref, b_ref, lab_ref,
               z_out, s1_out, sl_out):
  i = pl.program_id(0)

  @pl.when(i == 0)
  def _():
    z_out[...] = jnp.zeros_like(z_out)
    s1_out[...] = jnp.zeros_like(s1_out)
    sl_out[...] = jnp.zeros_like(sl_out)

  logits = (jnp.dot(visit_ref[...], w_ref[...],
                    preferred_element_type=jnp.float32) + b_ref[...])
  col = i * _VT + lax.broadcasted_iota(jnp.int32, (1, _VT), 1)
  valid = col < vocab
  lab = lab_ref[...].reshape(SEG, _VT)
  expl = jnp.where(valid, jnp.exp(logits), 0.0)
  s1 = jnp.where(valid, lab * logits, 0.0)
  sl = jnp.where(valid, lab, 0.0)
  z_out[...] += jnp.sum(expl, axis=1, keepdims=True)
  s1_out[...] += jnp.sum(s1, axis=1, keepdims=True)
  sl_out[...] += jnp.sum(sl, axis=1, keepdims=True)


def _loss_call(visit_bf, w_bf, b2, labels, vocab):
  num_tiles = pl.cdiv(vocab, _VT)
  one = jax.ShapeDtypeStruct((SEG, 1), jnp.float32)
  return pl.pallas_call(
      functools.partial(_loss_body, num_tiles, vocab),
      grid=(num_tiles,),
      in_specs=[
          pl.BlockSpec((SEG, D), lambda i: (0, 0)),
          pl.BlockSpec((D, _VT), lambda i: (0, i)),
          pl.BlockSpec((1, _VT), lambda i: (0, i)),
          pl.BlockSpec((B, T, _VT), lambda i: (0, 0, i)),
      ],
      out_specs=(
          pl.BlockSpec((SEG, 1), lambda i: (0, 0)),
          pl.BlockSpec((SEG, 1), lambda i: (0, 0)),
          pl.BlockSpec((SEG, 1), lambda i: (0, 0)),
      ),
      out_shape=(one, one, one),
  )(visit_bf, w_bf, b2, labels)


def _combine_body(z1, s11, sl1, z2, s12, sl2, out_ref):
  z = z1[...] + z2[...]
  s1 = s11[...] + s12[...]
  sl = sl1[...] + sl2[...]
  loss_rows = s1 - jnp.log(z) * sl
  out_ref[...] = -jnp.sum(loss_rows, axis=0, keepdims=True) / B


def _combine_call(pdx, prx):
  return pl.pallas_call(
      _combine_body,
      out_shape=jax.ShapeDtypeStruct((1, 1), jnp.float32),
  )(*pdx, *prx)


def kernel(dxseqs, drugseqs, dx_onehot, drug_onehot, EHRdxEmb, EHRdrugEmb,
           attn_W, attn_b, attnC_W, attnC_b, dp_W, dp_b, read_W, read_b,
           mort_W, mort_b, co_W, co_b):
  dxi = dxseqs.reshape(-1).astype(jnp.int32)
  rxi = drugseqs.reshape(-1).astype(jnp.int32)
  pooled = _bag(dxi, rxi, EHRdxEmb, EHRdrugEmb)

  visit_bf, dpPred, readPred, mortPred = _attn_call(
      pooled, attn_W, attn_b.reshape(1, ATTN), attnC_W.reshape(1, ATTN),
      attnC_b.reshape(1, 1), dp_W, dp_b.reshape(1, DPL),
      read_W.reshape(1, D), read_b.reshape(1, 1), mort_W.reshape(1, D),
      mort_b.reshape(1, 1))

  co_Wb = co_W.astype(jnp.bfloat16)
  pdx = _loss_call(visit_bf, co_Wb[:, :DXV], co_b[:DXV].reshape(1, DXV),
                   dx_onehot, DXV)
  prx = _loss_call(visit_bf, co_Wb[:, DXV:], co_b[DXV:].reshape(1, RXV),
                   drug_onehot, RXV)
  co_loss = _combine_call(pdx, prx)[0, 0]
  return dpPred, readPred, mortPred, co_loss


# speedup vs baseline: 1.0419x; 1.0086x over previous
import functools

import jax
import jax.numpy as jnp
from jax import lax
from jax.experimental import pallas as pl
from jax.experimental.pallas import tpu as pltpu
from jax.experimental.pallas import tpu_sc as plsc

B, T, DXN, RXN = 128, 20, 40, 40
D = 256
DXV, RXV = 4880, 3000
ATTN = 128
DPL = 4880
SEG = B * T

NC, NS = 2, 16
NW = NC * NS
SEGW = SEG // NW


_NCHUNK = SEGW
_NBUF = 4


def _make_bag():
  mesh = plsc.VectorSubcoreMesh(core_axis_name="c", subcore_axis_name="s")

  @functools.partial(
      pl.kernel,
      mesh=mesh,
      out_type=jax.ShapeDtypeStruct((SEG, D), jnp.float32),
      scratch_types=[
          pltpu.VMEM((SEGW * DXN,), jnp.int32),
          pltpu.VMEM((SEGW * RXN,), jnp.int32),
          pltpu.VMEM((_NBUF, DXN, D), jnp.float32),
          pltpu.VMEM((_NBUF, RXN, D), jnp.float32),
          pltpu.VMEM((SEGW, D), jnp.float32),
      ] + [pltpu.SemaphoreType.DMA] * (2 * _NBUF),
  )
  def bag(dxi_hbm, rxi_hbm, dxemb_hbm, rxemb_hbm, out_hbm,
          dxi_v, rxi_v, rows_dx, rows_rx, acc_v, *sems_flat):
    wid = lax.axis_index("s") * NC + lax.axis_index("c")
    base = wid * SEGW
    pltpu.sync_copy(dxi_hbm.at[pl.ds(base * DXN, SEGW * DXN)], dxi_v)
    pltpu.sync_copy(rxi_hbm.at[pl.ds(base * RXN, SEGW * RXN)], rxi_v)
    sems = tuple(
        (sems_flat[2 * u], sems_flat[2 * u + 1]) for u in range(_NBUF))

    def issue(c, slot):
      sd, sr = sems[slot]
      pltpu.async_copy(
          dxemb_hbm.at[dxi_v.at[pl.ds(c * DXN, DXN)]], rows_dx.at[slot], sd)
      pltpu.async_copy(
          rxemb_hbm.at[rxi_v.at[pl.ds(c * RXN, RXN)]], rows_rx.at[slot], sr)

    def consume(c, slot):
      def row_body(h, acc):
        r = 2 * h
        return tuple(
            acc[j]
            + (rows_dx[slot, r, pl.ds(16 * j, 16)]
               + rows_rx[slot, r, pl.ds(16 * j, 16)])
            + (rows_dx[slot, r + 1, pl.ds(16 * j, 16)]
               + rows_rx[slot, r + 1, pl.ds(16 * j, 16)])
            for j in range(D // 16))

      zeros = tuple(jnp.zeros((16,), jnp.float32) for _ in range(D // 16))
      acc = lax.fori_loop(0, DXN // 2, row_body, zeros)
      for j in range(D // 16):
        acc_v[c, pl.ds(16 * j, 16)] = acc[j]

    def drain(slot):
      sd, sr = sems[slot]
      pltpu.make_async_copy(
          dxemb_hbm.at[dxi_v.at[pl.ds(0, DXN)]], rows_dx.at[slot], sd).wait()
      pltpu.make_async_copy(
          rxemb_hbm.at[rxi_v.at[pl.ds(0, RXN)]], rows_rx.at[slot], sr).wait()

    for u in range(_NBUF - 1):
      issue(u, u)

    def ring_body(i, carry):
      c0 = _NBUF * i
      for u in range(_NBUF):
        c = c0 + u
        drain(u)
        consume(c, u)
        nxt = c + _NBUF - 1

        @pl.when(nxt < _NCHUNK)
        def _():
          issue(nxt, (u + _NBUF - 1) % _NBUF)

      return carry

    lax.fori_loop(0, _NCHUNK // _NBUF, ring_body, 0)
    pltpu.sync_copy(acc_v, out_hbm.at[pl.ds(base, SEGW)])

  return bag


_bag = _make_bag()


def _attn_body(pooled_ref, attn_w_ref, attn_b_ref, attnc_w_ref, attnc_b_ref,
               dp_w_ref, dp_b_ref, read_w_ref, read_b_ref, mort_w_ref,
               mort_b_ref, visit_out, dp_out, read_out, mort_out):
  visit = jnp.tanh(pooled_ref[...].astype(jnp.float32))
  v3 = visit.reshape(B, T, D)
  last = v3[:, T - 1, :]
  w1 = attn_w_ref[0:D, :]
  w2 = attn_w_ref[D:2 * D, :]
  h = jnp.dot(visit, w1, preferred_element_type=jnp.float32)
  h2 = jnp.dot(last, w2, preferred_element_type=jnp.float32)
  e = jnp.tanh(h.reshape(B, T, ATTN) + h2[:, None, :] + attn_b_ref[...])
  sc = jnp.sum(e * attnc_w_ref[...][None, :, :], axis=-1) + attnc_b_ref[0, 0]
  m = jnp.max(sc, axis=1, keepdims=True)
  a = jnp.exp(sc - m)
  alpha = a / jnp.sum(a, axis=1, keepdims=True)
  pt = jnp.sum(alpha[:, :, None] * v3, axis=1)
  dp = jax.nn.sigmoid(
      jnp.dot(pt, dp_w_ref[...], preferred_element_type=jnp.float32)
      + dp_b_ref[...])
  rd = jax.nn.sigmoid(
      jnp.sum(pt * read_w_ref[...], axis=-1, keepdims=True) + read_b_ref[0, 0])
  mt = jax.nn.sigmoid(
      jnp.sum(pt * mort_w_ref[...], axis=-1, keepdims=True) + mort_b_ref[0, 0])
  visit_out[...] = visit.astype(jnp.bfloat16)
  dp_out[...] = dp
  read_out[...] = rd
  mort_out[...] = mt


def _attn_call(pooled, attn_w, attn_b, attnc_w, attnc_b, dp_w, dp_b,
               read_w, read_b, mort_w, mort_b):
  return pl.pallas_call(
      _attn_body,
      out_shape=(
          jax.ShapeDtypeStruct((SEG, D), jnp.bfloat16),
          jax.ShapeDtypeStruct((B, DPL), jnp.float32),
          jax.ShapeDtypeStruct((B, 1), jnp.float32),
          jax.ShapeDtypeStruct((B, 1), jnp.float32),
      ),
  )(pooled, attn_w, attn_b, attnc_w, attnc_b, dp_w, dp_b, read_w, read_b,
    mort_w, mort_b)


_VT = 512


def _loss_body(num_tiles, vocab, visit_ref, w_ref, b_ref, lab_ref,
               z_out, s1_out, sl_out):
  i = pl.program_id(0)

  @pl.when(i == 0)
  def _():
    z_out[...] = jnp.zeros_like(z_out)
    s1_out[...] = jnp.zeros_like(s1_out)
    sl_out[...] = jnp.zeros_like(sl_out)

  logits = (jnp.dot(visit_ref[...], w_ref[...],
                    preferred_element_type=jnp.float32) + b_ref[...])
  col = i * _VT + lax.broadcasted_iota(jnp.int32, (1, _VT), 1)
  valid = col < vocab
  lab = lab_ref[...].reshape(SEG, _VT)
  expl = jnp.where(valid, jnp.exp(logits), 0.0)
  s1 = jnp.where(valid, lab * logits, 0.0)
  sl = jnp.where(valid, lab, 0.0)
  z_out[...] += jnp.sum(expl, axis=1, keepdims=True)
  s1_out[...] += jnp.sum(s1, axis=1, keepdims=True)
  sl_out[...] += jnp.sum(sl, axis=1, keepdims=True)


def _loss_call(visit_bf, w_bf, b2, labels, vocab):
  num_tiles = pl.cdiv(vocab, _VT)
  one = jax.ShapeDtypeStruct((SEG, 1), jnp.float32)
  return pl.pallas_call(
      functools.partial(_loss_body, num_tiles, vocab),
      grid=(num_tiles,),
      in_specs=[
          pl.BlockSpec((SEG, D), lambda i: (0, 0)),
          pl.BlockSpec((D, _VT), lambda i: (0, i)),
          pl.BlockSpec((1, _VT), lambda i: (0, i)),
          pl.BlockSpec((B, T, _VT), lambda i: (0, 0, i)),
      ],
      out_specs=(
          pl.BlockSpec((SEG, 1), lambda i: (0, 0)),
          pl.BlockSpec((SEG, 1), lambda i: (0, 0)),
          pl.BlockSpec((SEG, 1), lambda i: (0, 0)),
      ),
      out_shape=(one, one, one),
  )(visit_bf, w_bf, b2, labels)


def _combine_body(z1, s11, sl1, z2, s12, sl2, out_ref):
  z = z1[...] + z2[...]
  s1 = s11[...] + s12[...]
  sl = sl1[...] + sl2[...]
  loss_rows = s1 - jnp.log(z) * sl
  out_ref[...] = -jnp.sum(loss_rows, axis=0, keepdims=True) / B


def _combine_call(pdx, prx):
  return pl.pallas_call(
      _combine_body,
      out_shape=jax.ShapeDtypeStruct((1, 1), jnp.float32),
  )(*pdx, *prx)


def kernel(dxseqs, drugseqs, dx_onehot, drug_onehot, EHRdxEmb, EHRdrugEmb,
           attn_W, attn_b, attnC_W, attnC_b, dp_W, dp_b, read_W, read_b,
           mort_W, mort_b, co_W, co_b):
  dxi = dxseqs.reshape(-1).astype(jnp.int32)
  rxi = drugseqs.reshape(-1).astype(jnp.int32)
  pooled = _bag(dxi, rxi, EHRdxEmb, EHRdrugEmb)

  visit_bf, dpPred, readPred, mortPred = _attn_call(
      pooled, attn_W, attn_b.reshape(1, ATTN), attnC_W.reshape(1, ATTN),
      attnC_b.reshape(1, 1), dp_W, dp_b.reshape(1, DPL),
      read_W.reshape(1, D), read_b.reshape(1, 1), mort_W.reshape(1, D),
      mort_b.reshape(1, 1))

  co_Wb = co_W.astype(jnp.bfloat16)
  pdx = _loss_call(visit_bf, co_Wb[:, :DXV], co_b[:DXV].reshape(1, DXV),
                   dx_onehot, DXV)
  prx = _loss_call(visit_bf, co_Wb[:, DXV:], co_b[DXV:].reshape(1, RXV),
                   drug_onehot, RXV)
  co_loss = _combine_call(pdx, prx)[0, 0]
  return dpPred, readPred, mortPred, co_loss
